# baseline (speedup 1.0000x reference)
import jax
import jax.numpy as jnp
import numpy as np
from jax import lax
from jax.experimental import pallas as pl
from jax.experimental.pallas import tpu as pltpu
from jax.experimental.pallas import tpu_sc as plsc

B = 16
N = 2048
L = 16
QV = 4
NCHUNK = N // L

def _chamfer_body(qh, rh, oh, qv, rv, pk, qnv, outv):
    c = lax.axis_index("c")
    s = lax.axis_index("s")
    w = c * 16 + s

    pltpu.sync_copy(qh.at[w], qv)
    pltpu.sync_copy(rh.at[w], rv)

    def prep_r(j, carry):
        o = pl.ds(j * L, L)
        rx = rv[0, o]
        ry = rv[1, o]
        rz = rv[2, o]
        m2 = jnp.float32(-2.0)
        pk[0, o] = m2 * rx
        pk[1, o] = m2 * ry
        pk[2, o] = m2 * rz
        pk[3, o] = rx * rx + ry * ry + rz * rz
        return carry

    lax.fori_loop(0, NCHUNK, prep_r, 0)

    def prep_q(j, carry):
        o = pl.ds(j * L, L)
        qx = qv[0, o]
        qy = qv[1, o]
        qz = qv[2, o]
        qnv[o] = qx * qx + qy * qy + qz * qz
        return carry

    lax.fori_loop(0, NCHUNK, prep_q, 0)

    big = jnp.full((L,), 3.0e38, dtype=jnp.float32)

    def qblock(ib, carry):
        os = [pl.ds((ib * QV + v) * L, L) for v in range(QV)]
        qx = [qv[0, o] for o in os]
        qy = [qv[1, o] for o in os]
        qz = [qv[2, o] for o in os]

        def jstep(j, ms):
            o = pl.ds(j * L, L)
            ax = pk[0, o]
            ay = pk[1, o]
            az = pk[2, o]
            sq = pk[3, o]
            ms = list(ms)
            for jj in range(L):
                bx = ax[jj]
                by = ay[jj]
                bz = az[jj]
                bq = sq[jj]
                for v in range(QV):
                    t = bq + qx[v] * bx + qy[v] * by + qz[v] * bz
                    ms[v] = jnp.minimum(ms[v], t)
            return tuple(ms)

        ms = lax.fori_loop(0, NCHUNK, jstep, (big,) * QV)
        for v in range(QV):
            outv[os[v]] = ms[v] + qnv[os[v]]
        return carry

    lax.fori_loop(0, N // (L * QV), qblock, 0)

    pltpu.sync_copy(outv, oh.at[w])


@jax.jit
def kernel(input1, input2):
    a = jnp.transpose(input1, (0, 2, 1))
    b = jnp.transpose(input2, (0, 2, 1))
    q = jnp.concatenate([a, b], axis=0)
    r = jnp.concatenate([b, a], axis=0)

    run = pl.kernel(
        _chamfer_body,
        out_type=jax.ShapeDtypeStruct((2 * B, N), jnp.float32),
        mesh=plsc.VectorSubcoreMesh(core_axis_name="c", subcore_axis_name="s"),
        scratch_types=[
            pltpu.VMEM((3, N), jnp.float32),
            pltpu.VMEM((3, N), jnp.float32),
            pltpu.VMEM((4, N), jnp.float32),
            pltpu.VMEM((N,), jnp.float32),
            pltpu.VMEM((N,), jnp.float32),
        ],
    )
    out = run(q, r)
    return (out[:B], out[B:])


# speedup vs baseline: 4.5668x; 4.5668x over previous
import jax
import jax.numpy as jnp
from jax import lax
from jax.experimental import pallas as pl
from jax.experimental.pallas import tpu as pltpu
from jax.experimental.pallas import tpu_sc as plsc

B = 16
N = 2048
L = 16
QB = 8
NCHUNK = N // L


def _chamfer_body(qh, rh, oh, qv, rv, sqv, outv):
    c = lax.axis_index("c")
    s = lax.axis_index("s")
    w = c * 16 + s

    pltpu.sync_copy(qh.at[w], qv)
    pltpu.sync_copy(rh.at[w], rv)

    def sq_step(j, carry):
        o = pl.ds(j * L, L)
        rx = rv[0, o]
        ry = rv[1, o]
        rz = rv[2, o]
        sqv[o] = rx * rx + ry * ry + rz * rz
        return carry

    lax.fori_loop(0, NCHUNK, sq_step, 0)

    big = jnp.full((L,), 3.0e38, dtype=jnp.float32)
    lane = lax.iota(jnp.int32, L)

    def qblock(ib, carry):
        o = pl.ds(ib * L, L)
        qxv = qv[0, o]
        qyv = qv[1, o]
        qzv = qv[2, o]
        axv = jnp.float32(-2.0) * qxv
        ayv = jnp.float32(-2.0) * qyv
        azv = jnp.float32(-2.0) * qzv
        qn = qxv * qxv + qyv * qyv + qzv * qzv

        res = big
        for half in range(L // QB):
            ax = [axv[half * QB + k] for k in range(QB)]
            ay = [ayv[half * QB + k] for k in range(QB)]
            az = [azv[half * QB + k] for k in range(QB)]

            def jstep(j, ms):
                jo = pl.ds(j * L, L)
                rx = rv[0, jo]
                ry = rv[1, jo]
                rz = rv[2, jo]
                sq = sqv[jo]
                return tuple(
                    jnp.minimum(ms[k],
                                sq + ax[k] * rx + ay[k] * ry + az[k] * rz)
                    for k in range(QB)
                )

            ms = lax.fori_loop(0, NCHUNK, jstep, (big,) * QB)
            for k in range(QB):
                m = ms[k]
                for sh in (8, 4, 2, 1):
                    m = jnp.minimum(m, m[lane ^ sh])
                res = jnp.where(lane == half * QB + k, m, res)
        outv[o] = res + qn
        return carry

    lax.fori_loop(0, N // L, qblock, 0)

    pltpu.sync_copy(outv, oh.at[w])


@jax.jit
def kernel(input1, input2):
    a = jnp.transpose(input1, (0, 2, 1))
    b = jnp.transpose(input2, (0, 2, 1))
    q = jnp.concatenate([a, b], axis=0)
    r = jnp.concatenate([b, a], axis=0)

    run = pl.kernel(
        _chamfer_body,
        out_type=jax.ShapeDtypeStruct((2 * B, N), jnp.float32),
        mesh=plsc.VectorSubcoreMesh(core_axis_name="c", subcore_axis_name="s"),
        scratch_types=[
            pltpu.VMEM((3, N), jnp.float32),
            pltpu.VMEM((3, N), jnp.float32),
            pltpu.VMEM((N,), jnp.float32),
            pltpu.VMEM((N,), jnp.float32),
        ],
    )
    out = run(q, r)
    return (out[:B], out[B:])
